# baseline (speedup 1.0000x reference)
import jax
import jax.numpy as jnp
from jax.experimental import pallas as pl

K = 1024
D = 64
N = 32768
BLK = 256
NB = N // BLK


def _kmeans_step_body(x_ref, ct_ref, sums_ref, counts_ref):
    i = pl.program_id(0)
    x = x_ref[...]
    ct = ct_ref[...]
    cnorm = jnp.sum(ct * ct, axis=0)
    scores = jax.lax.dot_general(
        x, ct, (((1,), (0,)), ((), ())),
        preferred_element_type=jnp.float32,
        precision=jax.lax.Precision.HIGHEST,
    )
    dist = cnorm[None, :] - 2.0 * scores
    idx = jnp.argmin(dist, axis=1).astype(jnp.int32)
    onehot_t = (idx[None, :] ==
                jax.lax.broadcasted_iota(jnp.int32, (K, BLK), 0)
                ).astype(jnp.float32)
    part_sums = jax.lax.dot_general(
        onehot_t, x, (((1,), (0,)), ((), ())),
        preferred_element_type=jnp.float32,
        precision=jax.lax.Precision.HIGHEST,
    )
    part_counts = jnp.sum(onehot_t, axis=1)[None, :]

    @pl.when(i == 0)
    def _init():
        sums_ref[...] = part_sums
        counts_ref[...] = part_counts

    @pl.when(i > 0)
    def _acc():
        sums_ref[...] += part_sums
        counts_ref[...] += part_counts


@jax.jit
def _kmeans_step(x, ct):
    sums, counts = pl.pallas_call(
        _kmeans_step_body,
        grid=(NB,),
        in_specs=[
            pl.BlockSpec((BLK, D), lambda i: (i, 0)),
            pl.BlockSpec((D, K), lambda i: (0, 0)),
        ],
        out_specs=[
            pl.BlockSpec((K, D), lambda i: (0, 0)),
            pl.BlockSpec((1, K), lambda i: (0, 0)),
        ],
        out_shape=[
            jax.ShapeDtypeStruct((K, D), jnp.float32),
            jax.ShapeDtypeStruct((1, K), jnp.float32),
        ],
    )(x, ct)
    return sums, counts


def kernel(locF, Ck):
    x = locF.reshape(N, D)
    ct = Ck.reshape(K, D).T
    sums, counts = _kmeans_step(x, ct)
    Ck1 = sums[None, :, :]
    nItems = counts.astype(jnp.int64)
    return (Ck1, nItems)
